# Initial kernel scaffold; baseline (speedup 1.0000x reference)
import jax
import jax.numpy as jnp
from jax.experimental import pallas as pl


def kernel(x, pe):
    raise NotImplementedError("write your pallas kernel here")


# speedup vs baseline: 2.7229x; 2.7229x over previous
import functools

import numpy as np

import jax
import jax.numpy as jnp
from jax import lax
from jax.experimental import pallas as pl
from jax.experimental.pallas import tpu as pltpu
from jax.experimental.pallas import tpu_sc as plsc

RES = 100
D = 64
N_ROWS = 16384
N_COLS = 20
B = N_ROWS * N_COLS
NC = 2
NS = 16
NW = NC * NS
BPW = B // NW
C = 128
NCHUNK = BPW // C

_MAGIC = np.float32(2.0 ** 23)
_LO = np.float32(1.0 / RES)
_ONE = np.float32(1.0)
_RESF = np.float32(RES)


def _body(x_hbm, pe_hbm, out_hbm, x_v, idx_v, rows_v, sem):
    wid = lax.axis_index("s") * NC + lax.axis_index("c")
    base = wid * BPW

    def chunk(c, carry):
        off = base + c * C
        pltpu.sync_copy(x_hbm.at[pl.ds(off, C)], x_v)
        for i in range(C // 16):
            v = x_v[pl.ds(i * 16, 16)]
            xc = jnp.minimum(jnp.maximum(v, _LO), _ONE)
            r = (xc * _RESF + _MAGIC) - _MAGIC
            idx_v[pl.ds(i * 16, 16)] = (r - _ONE).astype(jnp.int32)
        pltpu.async_copy(pe_hbm.at[idx_v], rows_v, sem).wait()
        pltpu.sync_copy(rows_v, out_hbm.at[pl.ds(off, C)])
        return carry

    lax.fori_loop(0, NCHUNK, chunk, 0)


@jax.jit
def _emb(xf, pe):
    mesh = plsc.VectorSubcoreMesh(core_axis_name="c", subcore_axis_name="s")
    k = pl.kernel(
        _body,
        out_type=jax.ShapeDtypeStruct((B, D), jnp.float32),
        mesh=mesh,
        scratch_types=[
            pltpu.VMEM((C,), jnp.float32),
            pltpu.VMEM((C,), jnp.int32),
            pltpu.VMEM((C, D), jnp.float32),
            pltpu.SemaphoreType.DMA,
        ],
        compiler_params=pltpu.CompilerParams(use_tc_tiling_on_sc=False),
    )
    return k(xf, pe)


def kernel(x, pe):
    out = _emb(x.reshape(B), pe)
    return out.reshape(N_ROWS, N_COLS, D)


# speedup vs baseline: 2.7262x; 1.0012x over previous
import functools

import numpy as np

import jax
import jax.numpy as jnp
from jax import lax
from jax.experimental import pallas as pl
from jax.experimental.pallas import tpu as pltpu
from jax.experimental.pallas import tpu_sc as plsc

RES = 100
D = 64
N_ROWS = 16384
N_COLS = 20
B = N_ROWS * N_COLS
NC = 2
NS = 16
NW = NC * NS
BPW = B // NW
C = 128
NCHUNK = BPW // C
NBUF = 8
NGROUP = NCHUNK // NBUF
UNROLL = 8

_MAGIC = np.float32(2.0 ** 23)
_LO = np.float32(1.0 / RES)
_ONE = np.float32(1.0)
_RESF = np.float32(RES)


def _body(x_hbm, pe_hbm, out_hbm, x_v, idx_v, *rest):
    rows = rest[:NBUF]
    gsem = rest[NBUF:2 * NBUF]
    ssem = rest[2 * NBUF:3 * NBUF]
    wid = lax.axis_index("s") * NC + lax.axis_index("c")
    base = wid * BPW

    pltpu.sync_copy(x_hbm.at[pl.ds(base, BPW)], x_v)

    def cidx(i, carry):
        for u in range(UNROLL):
            o = i * (16 * UNROLL) + u * 16
            v = x_v[pl.ds(o, 16)]
            xc = jnp.minimum(jnp.maximum(v, _LO), _ONE)
            r = (xc * _RESF + _MAGIC) - _MAGIC
            idx_v[pl.ds(o, 16)] = (r - _ONE).astype(jnp.int32)
        return carry

    lax.fori_loop(0, BPW // (16 * UNROLL), cidx, 0)

    def fire_gather(c, b):
        return pltpu.async_copy(
            pe_hbm.at[idx_v.at[pl.ds(c * C, C)]], rows[b], gsem[b])

    def wait_gather(c, b):
        pltpu.make_async_copy(
            pe_hbm.at[idx_v.at[pl.ds(c * C, C)]], rows[b], gsem[b]).wait()

    def fire_scatter(c, b):
        return pltpu.async_copy(
            rows[b], out_hbm.at[pl.ds(base + c * C, C)], ssem[b])

    def wait_scatter(c, b):
        pltpu.make_async_copy(
            rows[b], out_hbm.at[pl.ds(base + c * C, C)], ssem[b]).wait()

    for b in range(NBUF):
        fire_gather(b, b)

    def group(g, carry):
        for b in range(NBUF):
            c = g * NBUF + b
            wait_gather(c, b)
            fire_scatter(c, b)
        for b in range(NBUF):
            c = g * NBUF + b
            wait_scatter(c, b)
            fire_gather(c + NBUF, b)
        return carry

    lax.fori_loop(0, NGROUP - 1, group, 0)

    for b in range(NBUF):
        c = (NGROUP - 1) * NBUF + b
        wait_gather(c, b)
        fire_scatter(c, b)
    for b in range(NBUF):
        c = (NGROUP - 1) * NBUF + b
        wait_scatter(c, b)


@jax.jit
def _emb(xf, pe):
    mesh = plsc.VectorSubcoreMesh(core_axis_name="c", subcore_axis_name="s")
    k = pl.kernel(
        _body,
        out_type=jax.ShapeDtypeStruct((B, D), jnp.float32),
        mesh=mesh,
        scratch_types=(
            [
                pltpu.VMEM((BPW,), jnp.float32),
                pltpu.VMEM((BPW,), jnp.int32),
            ]
            + [pltpu.VMEM((C, D), jnp.float32) for _ in range(NBUF)]
            + [pltpu.SemaphoreType.DMA for _ in range(2 * NBUF)]
        ),
        compiler_params=pltpu.CompilerParams(use_tc_tiling_on_sc=False),
    )
    return k(xf, pe)


def kernel(x, pe):
    out = _emb(x.reshape(B), pe)
    return out.reshape(N_ROWS, N_COLS, D)


# speedup vs baseline: 2.7286x; 1.0009x over previous
import functools

import numpy as np

import jax
import jax.numpy as jnp
from jax import lax
from jax.experimental import pallas as pl
from jax.experimental.pallas import tpu as pltpu
from jax.experimental.pallas import tpu_sc as plsc

RES = 100
D = 64
N_ROWS = 16384
N_COLS = 20
B = N_ROWS * N_COLS
NC = 2
NS = 16
NW = NC * NS
BPW = B // NW
C = 512
NCHUNK = BPW // C
NBUF = 2
NGROUP = NCHUNK // NBUF
UNROLL = 8

_MAGIC = np.float32(2.0 ** 23)
_LO = np.float32(1.0 / RES)
_ONE = np.float32(1.0)
_RESF = np.float32(RES)


def _body(x_hbm, pe_hbm, out_hbm, x_v, idx_v, *rest):
    rows = rest[:NBUF]
    gsem = rest[NBUF:2 * NBUF]
    ssem = rest[2 * NBUF:3 * NBUF]
    wid = lax.axis_index("s") * NC + lax.axis_index("c")
    base = wid * BPW

    pltpu.sync_copy(x_hbm.at[pl.ds(base, BPW)], x_v)

    def cidx(i, carry):
        for u in range(UNROLL):
            o = i * (16 * UNROLL) + u * 16
            v = x_v[pl.ds(o, 16)]
            xc = jnp.minimum(jnp.maximum(v, _LO), _ONE)
            r = (xc * _RESF + _MAGIC) - _MAGIC
            idx_v[pl.ds(o, 16)] = (r - _ONE).astype(jnp.int32)
        return carry

    lax.fori_loop(0, BPW // (16 * UNROLL), cidx, 0)

    def fire_gather(c, b):
        return pltpu.async_copy(
            pe_hbm.at[idx_v.at[pl.ds(c * C, C)]], rows[b], gsem[b])

    def wait_gather(c, b):
        pltpu.make_async_copy(
            pe_hbm.at[idx_v.at[pl.ds(c * C, C)]], rows[b], gsem[b]).wait()

    def fire_scatter(c, b):
        return pltpu.async_copy(
            rows[b], out_hbm.at[pl.ds(base + c * C, C)], ssem[b])

    def wait_scatter(c, b):
        pltpu.make_async_copy(
            rows[b], out_hbm.at[pl.ds(base + c * C, C)], ssem[b]).wait()

    for b in range(NBUF):
        fire_gather(b, b)

    def group(g, carry):
        for b in range(NBUF):
            c = g * NBUF + b
            wait_gather(c, b)
            fire_scatter(c, b)
        for b in range(NBUF):
            c = g * NBUF + b
            wait_scatter(c, b)
            fire_gather(c + NBUF, b)
        return carry

    lax.fori_loop(0, NGROUP - 1, group, 0)

    for b in range(NBUF):
        c = (NGROUP - 1) * NBUF + b
        wait_gather(c, b)
        fire_scatter(c, b)
    for b in range(NBUF):
        c = (NGROUP - 1) * NBUF + b
        wait_scatter(c, b)


@jax.jit
def _emb(xf, pe):
    mesh = plsc.VectorSubcoreMesh(core_axis_name="c", subcore_axis_name="s")
    k = pl.kernel(
        _body,
        out_type=jax.ShapeDtypeStruct((B, D), jnp.float32),
        mesh=mesh,
        scratch_types=(
            [
                pltpu.VMEM((BPW,), jnp.float32),
                pltpu.VMEM((BPW,), jnp.int32),
            ]
            + [pltpu.VMEM((C, D), jnp.float32) for _ in range(NBUF)]
            + [pltpu.SemaphoreType.DMA for _ in range(2 * NBUF)]
        ),
        compiler_params=pltpu.CompilerParams(use_tc_tiling_on_sc=False),
    )
    return k(xf, pe)


def kernel(x, pe):
    out = _emb(x.reshape(B), pe)
    return out.reshape(N_ROWS, N_COLS, D)
